# baseline (speedup 1.0000x reference)
import functools

import jax
import jax.numpy as jnp
from jax import lax
from jax.experimental import pallas as pl
from jax.experimental.pallas import tpu as pltpu
from jax.experimental.pallas import tpu_sc as plsc

NC = 2
NS = 16
NW = NC * NS
EB = 128

_mesh = plsc.VectorSubcoreMesh(core_axis_name="c", subcore_axis_name="s")


def _make_sc_agg(n, h, blocks, npad, bpp):
    zrows = npad // NS
    phases = blocks // bpp
    srows = bpp + 8

    @functools.partial(
        pl.kernel,
        out_type=jax.ShapeDtypeStruct((NC, npad, h), jnp.float32),
        mesh=_mesh,
        scratch_types=[
            pltpu.VMEM((srows, EB), jnp.int32),
            pltpu.VMEM((bpp, EB), jnp.int32),
            pltpu.VMEM((EB, h), jnp.float32),
            pltpu.VMEM((EB, h), jnp.float32),
            pltpu.VMEM_SHARED((npad, h), jnp.float32),
            pltpu.SemaphoreType.DMA,
            pltpu.SemaphoreType.DMA,
        ],
    )
    def sc_agg(h_hbm, src_hbm, dst_hbm, zeros_hbm, out_hbm,
               src_v, dst_v, rows0_v, rows1_v, agg_sh, sem0, sem1):
        cid = lax.axis_index("c")
        sid = lax.axis_index("s")
        wid = cid * NS + sid
        pltpu.sync_copy(zeros_hbm.at[pl.ds(sid * zrows, zrows)],
                        agg_sh.at[pl.ds(sid * zrows, zrows)])
        plsc.subcore_barrier()

        @pl.loop(0, phases)
        def _(ph):
            base = ph * bpp
            pltpu.sync_copy(src_hbm.at[wid, pl.ds(base, srows)], src_v)
            pltpu.sync_copy(dst_hbm.at[wid, pl.ds(base, bpp)], dst_v)

            @pl.loop(0, bpp)
            def _(b):
                pltpu.async_copy(h_hbm.at[src_v.at[b]], rows0_v, sem0).wait()
                pltpu.sync_copy(rows0_v, agg_sh.at[dst_v.at[b]], add=True)

        plsc.subcore_barrier()
        pltpu.sync_copy(agg_sh.at[pl.ds(sid * zrows, zrows)],
                        out_hbm.at[cid, pl.ds(sid * zrows, zrows)])

    return sc_agg


def _tc_layer_body(h_ref, p0_ref, p1_ref, w1_ref, b1_ref, g1_ref, bt1_ref,
                   w2_ref, b2_ref, o_ref):
    n = h_ref.shape[0]
    z = h_ref[...] + p0_ref[:n, :] + p1_ref[:n, :]
    z = jnp.dot(z, w1_ref[...], preferred_element_type=jnp.float32) + b1_ref[...]
    z = jnp.where(z >= 0, z, 0.2 * z)
    m = jnp.mean(z, axis=0)
    v = jnp.mean((z - m) ** 2, axis=0)
    z = (z - m) * lax.rsqrt(v + 1e-5) * g1_ref[...] + bt1_ref[...]
    z = jnp.dot(z, w2_ref[...], preferred_element_type=jnp.float32) + b2_ref[...]
    o_ref[...] = jnp.where(z >= 0, z, 0.2 * z)


def _tc_final_body(h_ref, batch_ref, g_ref, b_ref, fcw_ref, fcb_ref, o_ref):
    n, _ = h_ref.shape
    g = o_ref.shape[0]
    seg = lax.broadcasted_iota(jnp.int32, (g, n), 0)
    oh = (seg == batch_ref[...]).astype(jnp.float32)
    pooled = jnp.dot(oh, h_ref[...], preferred_element_type=jnp.float32)
    m = jnp.mean(pooled, axis=0)
    v = jnp.mean((pooled - m) ** 2, axis=0)
    pb = (pooled - m) * lax.rsqrt(v + 1e-5) * g_ref[...] + b_ref[...]
    o_ref[...] = jnp.dot(pb, fcw_ref[...],
                         preferred_element_type=jnp.float32) + fcb_ref[...]


def kernel(x, edge_index, batch, params):
    n, d = x.shape
    e = edge_index.shape[1]
    g = 16
    chunk = 2 * NW * EB
    epad = ((e + chunk - 1) // chunk) * chunk
    blocks = epad // (NW * EB)
    npad = ((n + 8 * NS) // (8 * NS)) * (8 * NS)

    src = edge_index[0]
    dst = edge_index[1]
    pad = epad - e
    src_p = jnp.concatenate(
        [src, jnp.zeros((pad,), jnp.int32)]).reshape(NW, blocks, EB)
    src_p = jnp.concatenate(
        [src_p, jnp.zeros((NW, 8, EB), jnp.int32)], axis=1)
    trash = jnp.asarray(n, jnp.int32) + jnp.arange(pad, dtype=jnp.int32) % (npad - n)
    dst_p = jnp.concatenate([dst, trash]).reshape(NW, blocks, EB)
    zeros_init = jnp.zeros((npad, d), jnp.float32)

    bpp = blocks // 2
    sc_agg = _make_sc_agg(n, d, blocks, npad, bpp)

    tc_layer = pl.pallas_call(
        _tc_layer_body,
        out_shape=jax.ShapeDtypeStruct((n, d), jnp.float32),
    )
    tc_final = pl.pallas_call(
        _tc_final_body,
        out_shape=jax.ShapeDtypeStruct((g, params['fc_W'].shape[1]),
                                       jnp.float32),
    )

    h = x
    for l in range(3):
        p = params['conv%d' % l]
        parts = sc_agg(h, src_p, dst_p, zeros_init)
        h = tc_layer(h, parts[0], parts[1], p['W1'], p['b1'], p['g1'],
                     p['bt1'], p['W2'], p['b2'])
    out = tc_final(h, batch.reshape(1, n).astype(jnp.int32),
                   params['bn_g'], params['bn_b'],
                   params['fc_W'], params['fc_b'])
    return out


# speedup vs baseline: 1.0026x; 1.0026x over previous
import functools

import jax
import jax.numpy as jnp
from jax import lax
from jax.experimental import pallas as pl
from jax.experimental.pallas import tpu as pltpu
from jax.experimental.pallas import tpu_sc as plsc

NC = 2
NS = 16
NW = NC * NS
EB = 128

_mesh = plsc.VectorSubcoreMesh(core_axis_name="c", subcore_axis_name="s")


def _make_sc_agg(n, h, blocks, npad, bpp):
    zrows = npad // NS

    @functools.partial(
        pl.kernel,
        out_type=jax.ShapeDtypeStruct((NC, npad, h), jnp.float32),
        mesh=_mesh,
        scratch_types=[
            pltpu.VMEM((blocks + 8, EB), jnp.int32),
            pltpu.VMEM((blocks, EB), jnp.int32),
            pltpu.VMEM((EB, h), jnp.float32),
            pltpu.VMEM_SHARED((npad, h), jnp.float32),
            pltpu.SemaphoreType.DMA,
        ],
    )
    def sc_agg(h_hbm, src_hbm, dst_hbm, zeros_hbm, out_hbm,
               src_v, dst_v, rows0_v, agg_sh, sem0):
        cid = lax.axis_index("c")
        sid = lax.axis_index("s")
        wid = cid * NS + sid
        pltpu.sync_copy(zeros_hbm.at[pl.ds(sid * zrows, zrows)],
                        agg_sh.at[pl.ds(sid * zrows, zrows)])
        pltpu.sync_copy(src_hbm.at[wid], src_v)
        pltpu.sync_copy(dst_hbm.at[wid], dst_v)
        plsc.subcore_barrier()

        @pl.loop(0, blocks)
        def _(b):
            pltpu.async_copy(h_hbm.at[src_v.at[b]], rows0_v, sem0).wait()
            pltpu.sync_copy(rows0_v, agg_sh.at[dst_v.at[b]], add=True)

        plsc.subcore_barrier()
        pltpu.sync_copy(agg_sh.at[pl.ds(sid * zrows, zrows)],
                        out_hbm.at[cid, pl.ds(sid * zrows, zrows)])

    return sc_agg


def _tc_layer_body(h_ref, p0_ref, p1_ref, w1_ref, b1_ref, g1_ref, bt1_ref,
                   w2_ref, b2_ref, o_ref):
    n = h_ref.shape[0]
    z = h_ref[...] + p0_ref[:n, :] + p1_ref[:n, :]
    z = jnp.dot(z, w1_ref[...], preferred_element_type=jnp.float32) + b1_ref[...]
    z = jnp.where(z >= 0, z, 0.2 * z)
    m = jnp.mean(z, axis=0)
    v = jnp.mean((z - m) ** 2, axis=0)
    z = (z - m) * lax.rsqrt(v + 1e-5) * g1_ref[...] + bt1_ref[...]
    z = jnp.dot(z, w2_ref[...], preferred_element_type=jnp.float32) + b2_ref[...]
    o_ref[...] = jnp.where(z >= 0, z, 0.2 * z)


def _tc_final_body(h_ref, batch_ref, g_ref, b_ref, fcw_ref, fcb_ref, o_ref):
    n, _ = h_ref.shape
    g = o_ref.shape[0]
    seg = lax.broadcasted_iota(jnp.int32, (g, n), 0)
    oh = (seg == batch_ref[...]).astype(jnp.float32)
    pooled = jnp.dot(oh, h_ref[...], preferred_element_type=jnp.float32)
    m = jnp.mean(pooled, axis=0)
    v = jnp.mean((pooled - m) ** 2, axis=0)
    pb = (pooled - m) * lax.rsqrt(v + 1e-5) * g_ref[...] + b_ref[...]
    o_ref[...] = jnp.dot(pb, fcw_ref[...],
                         preferred_element_type=jnp.float32) + fcb_ref[...]


def kernel(x, edge_index, batch, params):
    n, d = x.shape
    e = edge_index.shape[1]
    g = 16
    chunk = 2 * NW * EB
    epad = ((e + chunk - 1) // chunk) * chunk
    blocks = epad // (NW * EB)
    npad = ((n + 8 * NS) // (8 * NS)) * (8 * NS)

    src = edge_index[0]
    dst = edge_index[1]
    pad = epad - e
    src_p = jnp.concatenate(
        [src, jnp.zeros((pad,), jnp.int32)]).reshape(NW, blocks, EB)
    src_p = jnp.concatenate(
        [src_p, jnp.zeros((NW, 8, EB), jnp.int32)], axis=1)
    trash = jnp.asarray(n, jnp.int32) + jnp.arange(pad, dtype=jnp.int32) % (npad - n)
    dst_p = jnp.concatenate([dst, trash]).reshape(NW, blocks, EB)
    zeros_init = jnp.zeros((npad, d), jnp.float32)

    bpp = blocks // 2
    sc_agg = _make_sc_agg(n, d, blocks, npad, bpp)

    tc_layer = pl.pallas_call(
        _tc_layer_body,
        out_shape=jax.ShapeDtypeStruct((n, d), jnp.float32),
    )
    tc_final = pl.pallas_call(
        _tc_final_body,
        out_shape=jax.ShapeDtypeStruct((g, params['fc_W'].shape[1]),
                                       jnp.float32),
    )

    h = x
    for l in range(3):
        p = params['conv%d' % l]
        parts = sc_agg(h, src_p, dst_p, zeros_init)
        h = tc_layer(h, parts[0], parts[1], p['W1'], p['b1'], p['g1'],
                     p['bt1'], p['W2'], p['b2'])
    out = tc_final(h, batch.reshape(1, n).astype(jnp.int32),
                   params['bn_g'], params['bn_b'],
                   params['fc_W'], params['fc_b'])
    return out


# speedup vs baseline: 1.5647x; 1.5607x over previous
import functools

import jax
import jax.numpy as jnp
from jax import lax
from jax.experimental import pallas as pl
from jax.experimental.pallas import tpu as pltpu
from jax.experimental.pallas import tpu_sc as plsc

NC = 2
NS = 16
NW = NC * NS
EB = 128

_mesh = plsc.VectorSubcoreMesh(core_axis_name="c", subcore_axis_name="s")


def _make_sc_agg(n, h, blocks, npad, bpp):
    zrows = npad // NS

    @functools.partial(
        pl.kernel,
        out_type=jax.ShapeDtypeStruct((NC, npad, h), jnp.float32),
        mesh=_mesh,
        scratch_types=[
            pltpu.VMEM((blocks, EB), jnp.int32),
            pltpu.VMEM((blocks, EB), jnp.int32),
            pltpu.VMEM((EB, h), jnp.float32),
            pltpu.VMEM_SHARED((npad, h), jnp.float32),
            pltpu.SemaphoreType.DMA,
        ],
    )
    def sc_agg(h_hbm, src_hbm, dst_hbm, zeros_hbm, out_hbm,
               src_v, dst_v, rows0_v, agg_sh, sem0):
        cid = lax.axis_index("c")
        sid = lax.axis_index("s")
        wid = cid * NS + sid
        pltpu.sync_copy(zeros_hbm.at[pl.ds(sid * zrows, zrows)],
                        agg_sh.at[pl.ds(sid * zrows, zrows)])
        pltpu.sync_copy(src_hbm.at[wid], src_v)
        pltpu.sync_copy(dst_hbm.at[wid], dst_v)
        plsc.subcore_barrier()

        @pl.loop(0, blocks)
        def _(b):
            pltpu.async_copy(h_hbm.at[src_v.at[b]], rows0_v, sem0).wait()
            pltpu.sync_copy(rows0_v, agg_sh.at[dst_v.at[b]], add=True)

        plsc.subcore_barrier()
        pltpu.sync_copy(agg_sh.at[pl.ds(sid * zrows, zrows)],
                        out_hbm.at[cid, pl.ds(sid * zrows, zrows)])

    return sc_agg


def _tc_layer_body(h_ref, p0_ref, p1_ref, w1_ref, b1_ref, g1_ref, bt1_ref,
                   w2_ref, b2_ref, o_ref):
    n = h_ref.shape[0]
    z = h_ref[...] + p0_ref[:n, :] + p1_ref[:n, :]
    z = jnp.dot(z, w1_ref[...], preferred_element_type=jnp.float32) + b1_ref[...]
    z = jnp.where(z >= 0, z, 0.2 * z)
    m = jnp.mean(z, axis=0)
    v = jnp.mean((z - m) ** 2, axis=0)
    z = (z - m) * lax.rsqrt(v + 1e-5) * g1_ref[...] + bt1_ref[...]
    z = jnp.dot(z, w2_ref[...], preferred_element_type=jnp.float32) + b2_ref[...]
    o_ref[...] = jnp.where(z >= 0, z, 0.2 * z)


def _tc_final_body(h_ref, batch_ref, g_ref, b_ref, fcw_ref, fcb_ref, o_ref):
    n, _ = h_ref.shape
    g = o_ref.shape[0]
    seg = lax.broadcasted_iota(jnp.int32, (g, n), 0)
    oh = (seg == batch_ref[...]).astype(jnp.float32)
    pooled = jnp.dot(oh, h_ref[...], preferred_element_type=jnp.float32)
    m = jnp.mean(pooled, axis=0)
    v = jnp.mean((pooled - m) ** 2, axis=0)
    pb = (pooled - m) * lax.rsqrt(v + 1e-5) * g_ref[...] + b_ref[...]
    o_ref[...] = jnp.dot(pb, fcw_ref[...],
                         preferred_element_type=jnp.float32) + fcb_ref[...]


def kernel(x, edge_index, batch, params):
    n, d = x.shape
    e = edge_index.shape[1]
    g = 16
    chunk = NW * EB
    epad = ((e + chunk - 1) // chunk) * chunk
    blocks = epad // (NW * EB)
    npad = ((n + 8 * NS) // (8 * NS)) * (8 * NS)

    src = edge_index[0]
    dst = edge_index[1]
    pad = epad - e
    src_p = jnp.concatenate(
        [src, jnp.zeros((pad,), jnp.int32)]).reshape(NW, blocks, EB)
    dst_p = jnp.concatenate(
        [dst, jnp.full((pad,), n, jnp.int32)]).reshape(NW, blocks, EB)
    zeros_init = jnp.zeros((npad, d), jnp.float32)

    bpp = blocks // 2
    sc_agg = _make_sc_agg(n, d, blocks, npad, bpp)

    tc_layer = pl.pallas_call(
        _tc_layer_body,
        out_shape=jax.ShapeDtypeStruct((n, d), jnp.float32),
    )
    tc_final = pl.pallas_call(
        _tc_final_body,
        out_shape=jax.ShapeDtypeStruct((g, params['fc_W'].shape[1]),
                                       jnp.float32),
    )

    h = x
    for l in range(3):
        p = params['conv%d' % l]
        parts = sc_agg(h, src_p, dst_p, zeros_init)
        h = tc_layer(h, parts[0], parts[1], p['W1'], p['b1'], p['g1'],
                     p['bt1'], p['W2'], p['b2'])
    out = tc_final(h, batch.reshape(1, n).astype(jnp.int32),
                   params['bn_g'], params['bn_b'],
                   params['fc_W'], params['fc_b'])
    return out


# speedup vs baseline: 2.8185x; 1.8013x over previous
import functools

import jax
import jax.numpy as jnp
from jax import lax
from jax.experimental import pallas as pl
from jax.experimental.pallas import tpu as pltpu
from jax.experimental.pallas import tpu_sc as plsc

NC = 2
NS = 16
NW = NC * NS
EB = 128

_mesh = plsc.VectorSubcoreMesh(core_axis_name="c", subcore_axis_name="s")


def _make_sc_agg(n, h, blocks, npad, bpp):
    zrows = npad // NS

    @functools.partial(
        pl.kernel,
        out_type=jax.ShapeDtypeStruct((NC, npad, h), jnp.float32),
        mesh=_mesh,
        scratch_types=[
            pltpu.VMEM((blocks, EB), jnp.int32),
            pltpu.VMEM((blocks, EB), jnp.int32),
            pltpu.VMEM((EB, h), jnp.float32),
            pltpu.VMEM_SHARED((npad, h), jnp.float32),
            pltpu.SemaphoreType.DMA,
        ],
    )
    def sc_agg(h_hbm, src_hbm, dst_hbm, zeros_hbm, out_hbm,
               src_v, dst_v, rows0_v, agg_sh, sem0):
        cid = lax.axis_index("c")
        sid = lax.axis_index("s")
        wid = cid * NS + sid
        pltpu.sync_copy(zeros_hbm.at[pl.ds(sid * zrows, zrows)],
                        agg_sh.at[pl.ds(sid * zrows, zrows)])
        pltpu.sync_copy(src_hbm.at[wid], src_v)
        pltpu.sync_copy(dst_hbm.at[wid], dst_v)
        plsc.subcore_barrier()

        @pl.loop(0, blocks)
        def _(b):
            pltpu.async_copy(h_hbm.at[src_v.at[b]], rows0_v, sem0).wait()
            pltpu.sync_copy(rows0_v, agg_sh.at[dst_v.at[b]], add=True)

        plsc.subcore_barrier()
        pltpu.sync_copy(agg_sh.at[pl.ds(sid * zrows, zrows)],
                        out_hbm.at[cid, pl.ds(sid * zrows, zrows)])

    return sc_agg


def _tc_layer_body(h_ref, p0_ref, p1_ref, w1_ref, b1_ref, g1_ref, bt1_ref,
                   w2_ref, b2_ref, o_ref):
    n = h_ref.shape[0]
    z = h_ref[...] + p0_ref[:n, :] + p1_ref[:n, :]
    z = jnp.dot(z, w1_ref[...], preferred_element_type=jnp.float32) + b1_ref[...]
    z = jnp.where(z >= 0, z, 0.2 * z)
    m = jnp.mean(z, axis=0)
    v = jnp.mean((z - m) ** 2, axis=0)
    z = (z - m) * lax.rsqrt(v + 1e-5) * g1_ref[...] + bt1_ref[...]
    z = jnp.dot(z, w2_ref[...], preferred_element_type=jnp.float32) + b2_ref[...]
    o_ref[...] = jnp.where(z >= 0, z, 0.2 * z)


def _tc_final_body(h_ref, batch_ref, g_ref, b_ref, fcw_ref, fcb_ref, o_ref):
    n, _ = h_ref.shape
    g = o_ref.shape[0]
    seg = lax.broadcasted_iota(jnp.int32, (g, n), 0)
    oh = (seg == batch_ref[...]).astype(jnp.float32)
    pooled = jnp.dot(oh, h_ref[...], preferred_element_type=jnp.float32)
    m = jnp.mean(pooled, axis=0)
    v = jnp.mean((pooled - m) ** 2, axis=0)
    pb = (pooled - m) * lax.rsqrt(v + 1e-5) * g_ref[...] + b_ref[...]
    o_ref[...] = jnp.dot(pb, fcw_ref[...],
                         preferred_element_type=jnp.float32) + fcb_ref[...]


def kernel(x, edge_index, batch, params):
    n, d = x.shape
    e = edge_index.shape[1]
    g = 16
    chunk = NW * EB
    epad = ((e + chunk - 1) // chunk) * chunk
    blocks = epad // (NW * EB)
    npad = ((n + 8 * NS) // (8 * NS)) * (8 * NS)

    src = edge_index[0]
    dst = edge_index[1]
    pad = epad - e
    pad_ids = jnp.arange(pad, dtype=jnp.int32)
    src_p = jnp.concatenate([src, pad_ids % n]).reshape(NW, blocks, EB)
    dst_p = jnp.concatenate(
        [dst, n + pad_ids % (npad - n)]).reshape(NW, blocks, EB)
    zeros_init = jnp.zeros((npad, d), jnp.float32)

    bpp = blocks // 2
    sc_agg = _make_sc_agg(n, d, blocks, npad, bpp)

    tc_layer = pl.pallas_call(
        _tc_layer_body,
        out_shape=jax.ShapeDtypeStruct((n, d), jnp.float32),
    )
    tc_final = pl.pallas_call(
        _tc_final_body,
        out_shape=jax.ShapeDtypeStruct((g, params['fc_W'].shape[1]),
                                       jnp.float32),
    )

    h = x
    for l in range(3):
        p = params['conv%d' % l]
        parts = sc_agg(h, src_p, dst_p, zeros_init)
        h = tc_layer(h, parts[0], parts[1], p['W1'], p['b1'], p['g1'],
                     p['bt1'], p['W2'], p['b2'])
    out = tc_final(h, batch.reshape(1, n).astype(jnp.int32),
                   params['bn_g'], params['bn_b'],
                   params['fc_W'], params['fc_b'])
    return out


# speedup vs baseline: 3.1586x; 1.1207x over previous
import functools

import jax
import jax.numpy as jnp
from jax import lax
from jax.experimental import pallas as pl
from jax.experimental.pallas import tpu as pltpu
from jax.experimental.pallas import tpu_sc as plsc

NC = 2
NS = 16
NW = NC * NS
EB = 128

_mesh = plsc.VectorSubcoreMesh(core_axis_name="c", subcore_axis_name="s")


def _make_sc_agg(n, h, blocks, npad, bpp):
    zrows = npad // NS
    phases = blocks // bpp

    @functools.partial(
        pl.kernel,
        out_type=jax.ShapeDtypeStruct((NC, npad, h), jnp.float32),
        mesh=_mesh,
        scratch_types=[
            pltpu.VMEM((bpp, EB), jnp.int32),
            pltpu.VMEM((bpp, EB), jnp.int32),
            pltpu.VMEM((EB, h), jnp.float32),
            pltpu.VMEM((EB, h), jnp.float32),
            pltpu.VMEM_SHARED((npad, h), jnp.float32),
            pltpu.SemaphoreType.DMA,
            pltpu.SemaphoreType.DMA,
        ],
    )
    def sc_agg(h_hbm, src_hbm, dst_hbm, zeros_hbm, out_hbm,
               src_v, dst_v, rows0_v, rows1_v, agg_sh, sem0, sem1):
        cid = lax.axis_index("c")
        sid = lax.axis_index("s")
        wid = cid * NS + sid
        pltpu.sync_copy(zeros_hbm.at[pl.ds(sid * zrows, zrows)],
                        agg_sh.at[pl.ds(sid * zrows, zrows)])
        plsc.subcore_barrier()

        @pl.loop(0, phases)
        def _(ph):
            base = ph * bpp
            pltpu.sync_copy(src_hbm.at[wid, pl.ds(base, bpp)], src_v)
            pltpu.sync_copy(dst_hbm.at[wid, pl.ds(base, bpp)], dst_v)

            @pl.loop(0, bpp // 2)
            def _(i):
                b = 2 * i
                cp0 = pltpu.async_copy(h_hbm.at[src_v.at[b]], rows0_v, sem0)
                cp1 = pltpu.async_copy(h_hbm.at[src_v.at[b + 1]], rows1_v,
                                       sem1)
                cp0.wait()
                pltpu.sync_copy(rows0_v, agg_sh.at[dst_v.at[b]], add=True)
                cp1.wait()
                pltpu.sync_copy(rows1_v, agg_sh.at[dst_v.at[b + 1]], add=True)

        plsc.subcore_barrier()
        pltpu.sync_copy(agg_sh.at[pl.ds(sid * zrows, zrows)],
                        out_hbm.at[cid, pl.ds(sid * zrows, zrows)])

    return sc_agg


def _tc_layer_body(h_ref, p0_ref, p1_ref, w1_ref, b1_ref, g1_ref, bt1_ref,
                   w2_ref, b2_ref, o_ref):
    n = h_ref.shape[0]
    z = h_ref[...] + p0_ref[:n, :] + p1_ref[:n, :]
    z = jnp.dot(z, w1_ref[...], preferred_element_type=jnp.float32) + b1_ref[...]
    z = jnp.where(z >= 0, z, 0.2 * z)
    m = jnp.mean(z, axis=0)
    v = jnp.mean((z - m) ** 2, axis=0)
    z = (z - m) * lax.rsqrt(v + 1e-5) * g1_ref[...] + bt1_ref[...]
    z = jnp.dot(z, w2_ref[...], preferred_element_type=jnp.float32) + b2_ref[...]
    o_ref[...] = jnp.where(z >= 0, z, 0.2 * z)


def _tc_final_body(h_ref, batch_ref, g_ref, b_ref, fcw_ref, fcb_ref, o_ref):
    n, _ = h_ref.shape
    g = o_ref.shape[0]
    seg = lax.broadcasted_iota(jnp.int32, (g, n), 0)
    oh = (seg == batch_ref[...]).astype(jnp.float32)
    pooled = jnp.dot(oh, h_ref[...], preferred_element_type=jnp.float32)
    m = jnp.mean(pooled, axis=0)
    v = jnp.mean((pooled - m) ** 2, axis=0)
    pb = (pooled - m) * lax.rsqrt(v + 1e-5) * g_ref[...] + b_ref[...]
    o_ref[...] = jnp.dot(pb, fcw_ref[...],
                         preferred_element_type=jnp.float32) + fcb_ref[...]


def kernel(x, edge_index, batch, params):
    n, d = x.shape
    e = edge_index.shape[1]
    g = 16
    chunk = 2 * NW * EB
    epad = ((e + chunk - 1) // chunk) * chunk
    blocks = epad // (NW * EB)
    npad = ((n + 8 * NS) // (8 * NS)) * (8 * NS)

    src = edge_index[0]
    dst = edge_index[1]
    pad = epad - e
    pad_ids = jnp.arange(pad, dtype=jnp.int32)
    src_p = jnp.concatenate([src, pad_ids % n]).reshape(NW, blocks, EB)
    dst_p = jnp.concatenate(
        [dst, n + pad_ids % (npad - n)]).reshape(NW, blocks, EB)
    zeros_init = jnp.zeros((npad, d), jnp.float32)

    bpp = blocks // 2
    sc_agg = _make_sc_agg(n, d, blocks, npad, bpp)

    tc_layer = pl.pallas_call(
        _tc_layer_body,
        out_shape=jax.ShapeDtypeStruct((n, d), jnp.float32),
    )
    tc_final = pl.pallas_call(
        _tc_final_body,
        out_shape=jax.ShapeDtypeStruct((g, params['fc_W'].shape[1]),
                                       jnp.float32),
    )

    h = x
    for l in range(3):
        p = params['conv%d' % l]
        parts = sc_agg(h, src_p, dst_p, zeros_init)
        h = tc_layer(h, parts[0], parts[1], p['W1'], p['b1'], p['g1'],
                     p['bt1'], p['W2'], p['b2'])
    out = tc_final(h, batch.reshape(1, n).astype(jnp.int32),
                   params['bn_g'], params['bn_b'],
                   params['fc_W'], params['fc_b'])
    return out


# speedup vs baseline: 3.1906x; 1.0101x over previous
import functools

import jax
import jax.numpy as jnp
from jax import lax
from jax.experimental import pallas as pl
from jax.experimental.pallas import tpu as pltpu
from jax.experimental.pallas import tpu_sc as plsc

NC = 2
NS = 16
NW = NC * NS
EB = 128

_mesh = plsc.VectorSubcoreMesh(core_axis_name="c", subcore_axis_name="s")


def _make_sc_agg(n, h, blocks, npad, bpp):
    zrows = npad // NS
    phases = blocks // bpp

    @functools.partial(
        pl.kernel,
        out_type=jax.ShapeDtypeStruct((NC, npad, h), jnp.float32),
        mesh=_mesh,
        scratch_types=[
            pltpu.VMEM((bpp, EB), jnp.int32),
            pltpu.VMEM((bpp, EB), jnp.int32),
            pltpu.VMEM((EB, h), jnp.float32),
            pltpu.VMEM((EB, h), jnp.float32),
            pltpu.VMEM_SHARED((npad, h), jnp.float32),
            pltpu.SemaphoreType.DMA,
            pltpu.SemaphoreType.DMA,
        ],
    )
    def sc_agg(h_hbm, src_hbm, dst_hbm, zeros_hbm, out_hbm,
               src_v, dst_v, rows0_v, rows1_v, agg_sh, sem0, sem1):
        cid = lax.axis_index("c")
        sid = lax.axis_index("s")
        wid = cid * NS + sid
        pltpu.sync_copy(zeros_hbm.at[pl.ds(sid * zrows, zrows)],
                        agg_sh.at[pl.ds(sid * zrows, zrows)])
        plsc.subcore_barrier()

        @pl.loop(0, phases)
        def _(ph):
            base = ph * bpp
            pltpu.sync_copy(src_hbm.at[wid, pl.ds(base, bpp)], src_v)
            pltpu.sync_copy(dst_hbm.at[wid, pl.ds(base, bpp)], dst_v)

            @pl.loop(0, bpp // 2)
            def _(i):
                b = 2 * i
                cp0 = pltpu.async_copy(h_hbm.at[src_v.at[b]], rows0_v, sem0)
                cp1 = pltpu.async_copy(h_hbm.at[src_v.at[b + 1]], rows1_v,
                                       sem1)
                cp0.wait()
                pltpu.sync_copy(rows0_v, agg_sh.at[dst_v.at[b]], add=True)
                cp1.wait()
                pltpu.sync_copy(rows1_v, agg_sh.at[dst_v.at[b + 1]], add=True)

        plsc.subcore_barrier()
        pltpu.sync_copy(agg_sh.at[pl.ds(sid * zrows, zrows)],
                        out_hbm.at[cid, pl.ds(sid * zrows, zrows)])

    return sc_agg


def _tc_layer_body(h_ref, p0_ref, p1_ref, w1_ref, b1_ref, g1_ref, bt1_ref,
                   w2_ref, b2_ref, o_ref):
    n = h_ref.shape[0]
    z = h_ref[...] + p0_ref[:n, :] + p1_ref[:n, :]
    z = jnp.dot(z, w1_ref[...], preferred_element_type=jnp.float32) + b1_ref[...]
    z = jnp.where(z >= 0, z, 0.2 * z)
    m = jnp.mean(z, axis=0)
    v = jnp.mean((z - m) ** 2, axis=0)
    z = (z - m) * lax.rsqrt(v + 1e-5) * g1_ref[...] + bt1_ref[...]
    z = jnp.dot(z, w2_ref[...], preferred_element_type=jnp.float32) + b2_ref[...]
    o_ref[...] = jnp.where(z >= 0, z, 0.2 * z)


def _tc_last_body(h_ref, p0_ref, p1_ref, w1_ref, b1_ref, g1_ref, bt1_ref,
                  w2_ref, b2_ref, batch_ref, g_ref, b_ref, fcw_ref, fcb_ref,
                  o_ref):
    n = h_ref.shape[0]
    z = h_ref[...] + p0_ref[:n, :] + p1_ref[:n, :]
    z = jnp.dot(z, w1_ref[...], preferred_element_type=jnp.float32) + b1_ref[...]
    z = jnp.where(z >= 0, z, 0.2 * z)
    m = jnp.mean(z, axis=0)
    v = jnp.mean((z - m) ** 2, axis=0)
    z = (z - m) * lax.rsqrt(v + 1e-5) * g1_ref[...] + bt1_ref[...]
    z = jnp.dot(z, w2_ref[...], preferred_element_type=jnp.float32) + b2_ref[...]
    z = jnp.where(z >= 0, z, 0.2 * z)
    g = o_ref.shape[0]
    seg = lax.broadcasted_iota(jnp.int32, (g, n), 0)
    oh = (seg == batch_ref[...]).astype(jnp.float32)
    pooled = jnp.dot(oh, z, preferred_element_type=jnp.float32)
    m = jnp.mean(pooled, axis=0)
    v = jnp.mean((pooled - m) ** 2, axis=0)
    pb = (pooled - m) * lax.rsqrt(v + 1e-5) * g_ref[...] + b_ref[...]
    o_ref[...] = jnp.dot(pb, fcw_ref[...],
                         preferred_element_type=jnp.float32) + fcb_ref[...]


def kernel(x, edge_index, batch, params):
    n, d = x.shape
    e = edge_index.shape[1]
    g = 16
    chunk = 2 * NW * EB
    epad = ((e + chunk - 1) // chunk) * chunk
    blocks = epad // (NW * EB)
    npad = ((n + 8 * NS) // (8 * NS)) * (8 * NS)

    src = edge_index[0]
    dst = edge_index[1]
    pad = epad - e
    pad_ids = jnp.arange(pad, dtype=jnp.int32)
    src_p = jnp.concatenate([src, pad_ids % n]).reshape(NW, blocks, EB)
    dst_p = jnp.concatenate(
        [dst, n + pad_ids % (npad - n)]).reshape(NW, blocks, EB)
    zeros_init = jnp.zeros((npad, d), jnp.float32)

    bpp = blocks // 2
    sc_agg = _make_sc_agg(n, d, blocks, npad, bpp)

    tc_layer = pl.pallas_call(
        _tc_layer_body,
        out_shape=jax.ShapeDtypeStruct((n, d), jnp.float32),
    )
    tc_last = pl.pallas_call(
        _tc_last_body,
        out_shape=jax.ShapeDtypeStruct((g, params['fc_W'].shape[1]),
                                       jnp.float32),
    )

    h = x
    for l in range(2):
        p = params['conv%d' % l]
        parts = sc_agg(h, src_p, dst_p, zeros_init)
        h = tc_layer(h, parts[0], parts[1], p['W1'], p['b1'], p['g1'],
                     p['bt1'], p['W2'], p['b2'])
    p = params['conv2']
    parts = sc_agg(h, src_p, dst_p, zeros_init)
    out = tc_last(h, parts[0], parts[1], p['W1'], p['b1'], p['g1'],
                  p['bt1'], p['W2'], p['b2'],
                  batch.reshape(1, n).astype(jnp.int32),
                  params['bn_g'], params['bn_b'],
                  params['fc_W'], params['fc_b'])
    return out


# speedup vs baseline: 3.2273x; 1.0115x over previous
import functools

import jax
import jax.numpy as jnp
from jax import lax
from jax.experimental import pallas as pl
from jax.experimental.pallas import tpu as pltpu
from jax.experimental.pallas import tpu_sc as plsc

NC = 2
NS = 16
NW = NC * NS
EB = 128

_mesh = plsc.VectorSubcoreMesh(core_axis_name="c", subcore_axis_name="s")


def _make_sc_agg(n, h, blocks, npad, bpp):
    zrows = npad // NS
    phases = blocks // bpp

    @functools.partial(
        pl.kernel,
        out_type=jax.ShapeDtypeStruct((NC, npad, h), jnp.float32),
        mesh=_mesh,
        scratch_types=[
            pltpu.VMEM((bpp, EB), jnp.int32),
            pltpu.VMEM((bpp, EB), jnp.int32),
            pltpu.VMEM((EB, h), jnp.float32),
            pltpu.VMEM((EB, h), jnp.float32),
            pltpu.VMEM_SHARED((npad, h), jnp.float32),
            pltpu.SemaphoreType.DMA,
            pltpu.SemaphoreType.DMA,
            pltpu.SemaphoreType.DMA,
            pltpu.SemaphoreType.DMA,
        ],
    )
    def sc_agg(h_hbm, src_hbm, dst_hbm, zeros_hbm, out_hbm,
               src_v, dst_v, rows0_v, rows1_v, agg_sh, sem0, sem1,
               sem2, sem3):
        cid = lax.axis_index("c")
        sid = lax.axis_index("s")
        wid = cid * NS + sid
        pltpu.sync_copy(zeros_hbm.at[pl.ds(sid * zrows, zrows)],
                        agg_sh.at[pl.ds(sid * zrows, zrows)])
        plsc.subcore_barrier()

        @pl.loop(0, phases)
        def _(ph):
            base = ph * bpp
            pltpu.sync_copy(src_hbm.at[wid, pl.ds(base, bpp)], src_v)
            pltpu.sync_copy(dst_hbm.at[wid, pl.ds(base, bpp)], dst_v)

            @pl.loop(0, bpp // 2)
            def _(i):
                b = 2 * i
                cp0 = pltpu.async_copy(h_hbm.at[src_v.at[b]], rows0_v, sem0)
                cp1 = pltpu.async_copy(h_hbm.at[src_v.at[b + 1]], rows1_v,
                                       sem1)
                cp0.wait()
                sc0 = pltpu.async_copy(rows0_v, agg_sh.at[dst_v.at[b]], sem2,
                                       add=True)
                cp1.wait()
                sc1 = pltpu.async_copy(rows1_v, agg_sh.at[dst_v.at[b + 1]],
                                       sem3, add=True)
                sc0.wait()
                sc1.wait()

        plsc.subcore_barrier()
        pltpu.sync_copy(agg_sh.at[pl.ds(sid * zrows, zrows)],
                        out_hbm.at[cid, pl.ds(sid * zrows, zrows)])

    return sc_agg


def _tc_layer_body(h_ref, p0_ref, p1_ref, w1_ref, b1_ref, g1_ref, bt1_ref,
                   w2_ref, b2_ref, o_ref):
    n = h_ref.shape[0]
    z = h_ref[...] + p0_ref[:n, :] + p1_ref[:n, :]
    z = jnp.dot(z, w1_ref[...], preferred_element_type=jnp.float32) + b1_ref[...]
    z = jnp.where(z >= 0, z, 0.2 * z)
    m = jnp.mean(z, axis=0)
    v = jnp.mean((z - m) ** 2, axis=0)
    z = (z - m) * lax.rsqrt(v + 1e-5) * g1_ref[...] + bt1_ref[...]
    z = jnp.dot(z, w2_ref[...], preferred_element_type=jnp.float32) + b2_ref[...]
    o_ref[...] = jnp.where(z >= 0, z, 0.2 * z)


def _tc_last_body(h_ref, p0_ref, p1_ref, w1_ref, b1_ref, g1_ref, bt1_ref,
                  w2_ref, b2_ref, batch_ref, g_ref, b_ref, fcw_ref, fcb_ref,
                  o_ref):
    n = h_ref.shape[0]
    z = h_ref[...] + p0_ref[:n, :] + p1_ref[:n, :]
    z = jnp.dot(z, w1_ref[...], preferred_element_type=jnp.float32) + b1_ref[...]
    z = jnp.where(z >= 0, z, 0.2 * z)
    m = jnp.mean(z, axis=0)
    v = jnp.mean((z - m) ** 2, axis=0)
    z = (z - m) * lax.rsqrt(v + 1e-5) * g1_ref[...] + bt1_ref[...]
    z = jnp.dot(z, w2_ref[...], preferred_element_type=jnp.float32) + b2_ref[...]
    z = jnp.where(z >= 0, z, 0.2 * z)
    g = o_ref.shape[0]
    seg = lax.broadcasted_iota(jnp.int32, (g, n), 0)
    oh = (seg == batch_ref[...]).astype(jnp.float32)
    pooled = jnp.dot(oh, z, preferred_element_type=jnp.float32)
    m = jnp.mean(pooled, axis=0)
    v = jnp.mean((pooled - m) ** 2, axis=0)
    pb = (pooled - m) * lax.rsqrt(v + 1e-5) * g_ref[...] + b_ref[...]
    o_ref[...] = jnp.dot(pb, fcw_ref[...],
                         preferred_element_type=jnp.float32) + fcb_ref[...]


def kernel(x, edge_index, batch, params):
    n, d = x.shape
    e = edge_index.shape[1]
    g = 16
    chunk = 2 * NW * EB
    epad = ((e + chunk - 1) // chunk) * chunk
    blocks = epad // (NW * EB)
    npad = ((n + 8 * NS) // (8 * NS)) * (8 * NS)

    src = edge_index[0]
    dst = edge_index[1]
    pad = epad - e
    pad_ids = jnp.arange(pad, dtype=jnp.int32)
    src_p = jnp.concatenate([src, pad_ids % n]).reshape(NW, blocks, EB)
    dst_p = jnp.concatenate(
        [dst, n + pad_ids % (npad - n)]).reshape(NW, blocks, EB)
    zeros_init = jnp.zeros((npad, d), jnp.float32)

    bpp = blocks // 2
    sc_agg = _make_sc_agg(n, d, blocks, npad, bpp)

    tc_layer = pl.pallas_call(
        _tc_layer_body,
        out_shape=jax.ShapeDtypeStruct((n, d), jnp.float32),
    )
    tc_last = pl.pallas_call(
        _tc_last_body,
        out_shape=jax.ShapeDtypeStruct((g, params['fc_W'].shape[1]),
                                       jnp.float32),
    )

    h = x
    for l in range(2):
        p = params['conv%d' % l]
        parts = sc_agg(h, src_p, dst_p, zeros_init)
        h = tc_layer(h, parts[0], parts[1], p['W1'], p['b1'], p['g1'],
                     p['bt1'], p['W2'], p['b2'])
    p = params['conv2']
    parts = sc_agg(h, src_p, dst_p, zeros_init)
    out = tc_last(h, parts[0], parts[1], p['W1'], p['b1'], p['g1'],
                  p['bt1'], p['W2'], p['b2'],
                  batch.reshape(1, n).astype(jnp.int32),
                  params['bn_g'], params['bn_b'],
                  params['fc_W'], params['fc_b'])
    return out
